# Initial kernel scaffold; baseline (speedup 1.0000x reference)
import jax
import jax.numpy as jnp
from jax.experimental import pallas as pl


def kernel(input_ids, embed_table, W, b):
    raise NotImplementedError("write your pallas kernel here")


# speedup vs baseline: 1.3104x; 1.3104x over previous
import functools

import jax
import jax.numpy as jnp
from jax import lax
from jax.experimental import pallas as pl
from jax.experimental.pallas import tpu as pltpu
from jax.experimental.pallas import tpu_sc as plsc

D_MODEL = 4096

_NC = 2
_NS = 16
_NW = _NC * _NS
_CHUNK = 16


def _sc_gather(table, ids3):
    n_ch = ids3.shape[1]
    rows_per_w = n_ch * _CHUNK
    total = _NW * rows_per_w
    d = table.shape[1]
    mesh = plsc.VectorSubcoreMesh(core_axis_name="c", subcore_axis_name="s")

    @functools.partial(
        pl.kernel,
        mesh=mesh,
        out_type=jax.ShapeDtypeStruct((total, d), jnp.float32),
        scratch_types=[
            pltpu.VMEM((n_ch, _CHUNK), jnp.int32),
            pltpu.VMEM((_CHUNK, d), jnp.float32),
            pltpu.SemaphoreType.DMA,
        ],
    )
    def gather_kernel(table_hbm, ids_hbm, out_hbm, idx_v, rows_v, sem):
        wid = lax.axis_index("s") * _NC + lax.axis_index("c")
        base = wid * rows_per_w
        pltpu.sync_copy(ids_hbm.at[wid], idx_v)

        def body(c, _):
            pltpu.async_copy(table_hbm.at[idx_v.at[c]], rows_v, sem).wait()
            pltpu.sync_copy(rows_v, out_hbm.at[pl.ds(base + c * _CHUNK, _CHUNK)])
            return _

        lax.fori_loop(0, n_ch, body, None)

    return gather_kernel(table, ids3)


def _tc_linear(x, w_bf16, b2):
    n, d = x.shape
    bt, bo = 1024, 512
    n_t, n_o = n // bt, d // bo

    def body(x_ref, w_ref, b_ref, o_ref, xb):
        @pl.when(pl.program_id(1) == 0)
        def _():
            xb[...] = x_ref[...].astype(jnp.bfloat16)

        acc = lax.dot_general(
            xb[...], w_ref[...], (((1,), (1,)), ((), ())),
            preferred_element_type=jnp.float32,
        )
        o_ref[...] = acc + b_ref[...]

    return pl.pallas_call(
        body,
        grid=(n_t, n_o),
        in_specs=[
            pl.BlockSpec((bt, d), lambda t, o: (t, 0)),
            pl.BlockSpec((bo, d), lambda t, o: (o, 0)),
            pl.BlockSpec((1, bo), lambda t, o: (0, o)),
        ],
        out_specs=pl.BlockSpec((bt, bo), lambda t, o: (t, o)),
        out_shape=jax.ShapeDtypeStruct((n, d), jnp.float32),
        scratch_shapes=[pltpu.VMEM((bt, d), jnp.bfloat16)],
    )(x, w_bf16, b2)


def kernel(input_ids, embed_table, W, b):
    batch, seq = input_ids.shape
    n = batch * seq
    ids3 = input_ids.reshape(_NW, n // (_NW * _CHUNK), _CHUNK).astype(jnp.int32)
    x = _sc_gather(embed_table, ids3)
    out = _tc_linear(x, W.astype(jnp.bfloat16), b.reshape(1, -1))
    return out.reshape(batch, seq, D_MODEL)
